# baseline (speedup 1.0000x reference)
import jax
import jax.numpy as jnp
from jax.experimental import pallas as pl


def _relu_block(x_ref, o_ref):
    o_ref[...] = jnp.maximum(x_ref[...], 0.0)


def kernel(input):
    m, n = input.shape
    block_m = 1024
    grid = (m // block_m,)
    return pl.pallas_call(
        _relu_block,
        grid=grid,
        in_specs=[pl.BlockSpec((block_m, n), lambda i: (i, 0))],
        out_specs=pl.BlockSpec((block_m, n), lambda i: (i, 0)),
        out_shape=jax.ShapeDtypeStruct((m, n), input.dtype),
    )(input)
